# Initial kernel scaffold; baseline (speedup 1.0000x reference)
import jax
import jax.numpy as jnp
from jax.experimental import pallas as pl


def kernel(atom_hiddens, a_scope):
    raise NotImplementedError("write your pallas kernel here")


# speedup vs baseline: 5.3692x; 5.3692x over previous
import functools

import jax
import jax.numpy as jnp
from jax import lax
from jax.experimental import pallas as pl
from jax.experimental.pallas import tpu as pltpu
from jax.experimental.pallas import tpu_sc as plsc

N = 32768
D = 256
B = 16
L = 16
NC, NS = 2, 16
NW = NC * NS
BLK = 64
NBLK = N // BLK
BLK_PER_W = NBLK // NW
ROWS_PER_W = N // NW
DH = D // 2
EDGE = 2 * BLK + 8

_mesh = plsc.VectorSubcoreMesh(core_axis_name="c", subcore_axis_name="s")


def _i32(v):
    return jnp.asarray(v, jnp.int32)


def _lane_i32(vec, m):
    mask = (lax.iota(jnp.int32, L) == m).astype(jnp.int32)
    return jnp.sum(vec * mask, dtype=jnp.int32)


def _lane_f32(vec, m):
    mask = (lax.iota(jnp.int32, L) == m).astype(jnp.float32)
    return jnp.sum(vec * mask, dtype=jnp.float32)


@functools.partial(
    pl.kernel,
    out_type=jax.ShapeDtypeStruct((NBLK, D), jnp.float32),
    mesh=_mesh,
    scratch_types=[
        pltpu.VMEM((2, BLK, D), jnp.float32),
        pltpu.VMEM((BLK_PER_W, D), jnp.float32),
        pltpu.SemaphoreType.DMA,
        pltpu.SemaphoreType.DMA,
    ],
)
def _block_sums(x_hbm, bs_hbm, buf, acc_v, sem0, sem1):
    wid = lax.axis_index("s") * NC + lax.axis_index("c")
    row0 = wid * ROWS_PER_W
    sems = (sem0, sem1)
    copies = [None, None]
    copies[0] = pltpu.async_copy(
        x_hbm.at[pl.ds(row0, BLK)], buf.at[_i32(0)], sem0)
    for b in range(BLK_PER_W):
        cur = b % 2
        if b + 1 < BLK_PER_W:
            nxt = (b + 1) % 2
            copies[nxt] = pltpu.async_copy(
                x_hbm.at[pl.ds(row0 + (b + 1) * BLK, BLK)], buf.at[_i32(nxt)],
                sems[nxt])
        copies[cur].wait()
        bb = buf.at[_i32(cur)]

        def body(r, accs):
            return tuple(accs[c] + bb[r, pl.ds(c * L, L)]
                         for c in range(D // L))

        accs = lax.fori_loop(
            0, BLK, body,
            tuple(jnp.zeros((L,), jnp.float32) for _ in range(D // L)))
        for c in range(D // L):
            acc_v[_i32(b), pl.ds(c * L, L)] = accs[c]
    pltpu.sync_copy(acc_v, bs_hbm.at[pl.ds(wid * BLK_PER_W, BLK_PER_W)])


@functools.partial(
    pl.kernel,
    out_type=jax.ShapeDtypeStruct((B * D,), jnp.float32),
    mesh=_mesh,
    scratch_types=[
        pltpu.VMEM((8, L), jnp.int32),
        pltpu.VMEM((L,), jnp.float32),
        pltpu.VMEM((NBLK, DH), jnp.float32),
        pltpu.VMEM((EDGE, DH), jnp.float32),
        pltpu.VMEM((EDGE, DH), jnp.float32),
        pltpu.VMEM((DH,), jnp.float32),
        pltpu.SemaphoreType.DMA,
        pltpu.SemaphoreType.DMA,
        pltpu.SemaphoreType.DMA,
    ],
    compiler_params=pltpu.CompilerParams(needs_layout_passes=False),
)
def _combine(x_hbm, bs_hbm, pi_hbm, inv_hbm, out_hbm,
             pv, invv, bsv, e1v, e2v, outv, sem_bs, sem_e1, sem_e2):
    wid = lax.axis_index("s") * NC + lax.axis_index("c")
    m = wid // 2
    h = wid % 2
    col0 = h * DH

    pltpu.sync_copy(pi_hbm, pv)
    pltpu.sync_copy(inv_hbm, invv)
    fb_lo = _lane_i32(pv[_i32(0)], m)
    fb_hi = _lane_i32(pv[_i32(1)], m)
    e1_lo = _lane_i32(pv[_i32(2)], m)
    e1_hi = _lane_i32(pv[_i32(3)], m)
    e2_lo = _lane_i32(pv[_i32(4)], m)
    e2_hi = _lane_i32(pv[_i32(5)], m)
    c1 = pl.multiple_of(_lane_i32(pv[_i32(6)], m), 8)
    c2 = pl.multiple_of(_lane_i32(pv[_i32(7)], m), 8)
    inv = _lane_f32(invv[...], m)

    cp_bs = pltpu.async_copy(
        bs_hbm.at[pl.ds(0, NBLK), pl.ds(col0, DH)], bsv, sem_bs)
    cp_e1 = pltpu.async_copy(
        x_hbm.at[pl.ds(c1, EDGE), pl.ds(col0, DH)], e1v, sem_e1)
    cp_e2 = pltpu.async_copy(
        x_hbm.at[pl.ds(c2, EDGE), pl.ds(col0, DH)], e2v, sem_e2)

    zero = tuple(jnp.zeros((L,), jnp.float32) for _ in range(DH // L))

    cp_bs.wait()

    def fb_body(bk, accs):
        return tuple(accs[c] + bsv[bk, pl.ds(c * L, L)]
                     for c in range(DH // L))

    accs = lax.fori_loop(fb_lo, fb_hi, fb_body, zero)

    cp_e1.wait()

    def e1_body(r, accs):
        return tuple(accs[c] + e1v[r, pl.ds(c * L, L)]
                     for c in range(DH // L))

    accs = lax.fori_loop(e1_lo - c1, e1_hi - c1, e1_body, accs)

    cp_e2.wait()

    def e2_body(r, accs):
        return tuple(accs[c] + e2v[r, pl.ds(c * L, L)]
                     for c in range(DH // L))

    accs = lax.fori_loop(e2_lo - c2, e2_hi - c2, e2_body, accs)

    for c in range(DH // L):
        outv[pl.ds(c * L, L)] = accs[c] * inv
    pltpu.sync_copy(outv, out_hbm.at[pl.ds(m * D + col0, DH)])


def kernel(atom_hiddens, a_scope):
    x = atom_hiddens.astype(jnp.float32)
    s = a_scope[:, 0].astype(jnp.int32)
    sz = a_scope[:, 1].astype(jnp.int32)
    e = jnp.minimum(s + sz, N)
    b0 = (s + BLK - 1) // BLK
    b1 = e // BLK
    has_full = b0 < b1
    fb_lo = jnp.where(has_full, b0, 0)
    fb_hi = jnp.where(has_full, b1, 0)
    e1_lo = s
    e1_hi = jnp.where(has_full, b0 * BLK, e)
    e2_lo = jnp.where(has_full, b1 * BLK, 0)
    e2_hi = jnp.where(has_full, e, 0)
    c1 = jnp.minimum((e1_lo // 8) * 8, N - EDGE)
    c2 = jnp.minimum((e2_lo // 8) * 8, N - EDGE)
    pi = jnp.stack([fb_lo, fb_hi, e1_lo, e1_hi, e2_lo, e2_hi, c1, c2])
    inv = jnp.where(sz > 0, 1.0 / jnp.maximum(sz, 1).astype(jnp.float32), 0.0)

    bs = _block_sums(x)
    return _combine(x, bs, pi, inv).reshape(B, D)


# speedup vs baseline: 5.5891x; 1.0409x over previous
import functools

import jax
import jax.numpy as jnp
from jax import lax
from jax.experimental import pallas as pl
from jax.experimental.pallas import tpu as pltpu
from jax.experimental.pallas import tpu_sc as plsc

N = 32768
D = 256
B = 16
L = 16
NC, NS = 2, 16
NW = NC * NS
BLK = 64
NBLK = N // BLK
BLK_PER_W = NBLK // NW
ROWS_PER_W = N // NW
DH = D // 2
EDGE = 2 * BLK + 8
CHUNK = 2 * BLK

_mesh = plsc.VectorSubcoreMesh(core_axis_name="c", subcore_axis_name="s")


def _i32(v):
    return jnp.asarray(v, jnp.int32)


def _lane_i32(vec, m):
    mask = (lax.iota(jnp.int32, L) == m).astype(jnp.int32)
    return jnp.sum(vec * mask, dtype=jnp.int32)


def _lane_f32(vec, m):
    mask = (lax.iota(jnp.int32, L) == m).astype(jnp.float32)
    return jnp.sum(vec * mask, dtype=jnp.float32)


@functools.partial(
    pl.kernel,
    out_type=jax.ShapeDtypeStruct((NBLK, D), jnp.float32),
    mesh=_mesh,
    scratch_types=[
        pltpu.VMEM((2, CHUNK, D), jnp.float32),
        pltpu.VMEM((BLK_PER_W, D), jnp.float32),
        pltpu.SemaphoreType.DMA,
        pltpu.SemaphoreType.DMA,
    ],
)
def _block_sums(x_hbm, bs_hbm, buf, acc_v, sem0, sem1):
    wid = lax.axis_index("s") * NC + lax.axis_index("c")
    row0 = wid * ROWS_PER_W
    sems = (sem0, sem1)
    copies = [None, None]
    CH = CHUNK
    NCH = ROWS_PER_W // CH
    copies[0] = pltpu.async_copy(
        x_hbm.at[pl.ds(row0, CH)], buf.at[_i32(0)], sem0)
    for g in range(NCH):
        cur = g % 2
        if g + 1 < NCH:
            nxt = (g + 1) % 2
            copies[nxt] = pltpu.async_copy(
                x_hbm.at[pl.ds(row0 + (g + 1) * CH, CH)], buf.at[_i32(nxt)],
                sems[nxt])
        copies[cur].wait()
        bb = buf.at[_i32(cur)]
        for sb in range(CH // BLK):

            def body(r, accs, _sb=sb):
                r2 = r + r + _sb * BLK
                r3 = r2 + 1
                accs = tuple(accs[c] + bb[r2, pl.ds(c * L, L)]
                             for c in range(D // L))
                return tuple(accs[c] + bb[r3, pl.ds(c * L, L)]
                             for c in range(D // L))

            accs = lax.fori_loop(
                _i32(0), _i32(BLK // 2), body,
                tuple(jnp.zeros((L,), jnp.float32) for _ in range(D // L)))
            b = g * (CH // BLK) + sb
            for c in range(D // L):
                acc_v[_i32(b), pl.ds(c * L, L)] = accs[c]
    pltpu.sync_copy(acc_v, bs_hbm.at[pl.ds(wid * BLK_PER_W, BLK_PER_W)])


@functools.partial(
    pl.kernel,
    out_type=jax.ShapeDtypeStruct((B * D,), jnp.float32),
    mesh=_mesh,
    scratch_types=[
        pltpu.VMEM((8, L), jnp.int32),
        pltpu.VMEM((L,), jnp.float32),
        pltpu.VMEM((NBLK, DH), jnp.float32),
        pltpu.VMEM((EDGE, DH), jnp.float32),
        pltpu.VMEM((EDGE, DH), jnp.float32),
        pltpu.VMEM((DH,), jnp.float32),
        pltpu.SemaphoreType.DMA,
        pltpu.SemaphoreType.DMA,
        pltpu.SemaphoreType.DMA,
    ],
    compiler_params=pltpu.CompilerParams(needs_layout_passes=False),
)
def _combine(x_hbm, bs_hbm, pi_hbm, inv_hbm, out_hbm,
             pv, invv, bsv, e1v, e2v, outv, sem_bs, sem_e1, sem_e2):
    wid = lax.axis_index("s") * NC + lax.axis_index("c")
    m = wid // 2
    h = wid % 2
    col0 = h * DH

    cp_bs = pltpu.async_copy(
        bs_hbm.at[pl.ds(0, NBLK), pl.ds(col0, DH)], bsv, sem_bs)
    pltpu.sync_copy(pi_hbm, pv)
    pltpu.sync_copy(inv_hbm, invv)
    fb_lo = _lane_i32(pv[_i32(0)], m)
    fb_hi = _lane_i32(pv[_i32(1)], m)
    e1_lo = _lane_i32(pv[_i32(2)], m)
    e1_hi = _lane_i32(pv[_i32(3)], m)
    e2_lo = _lane_i32(pv[_i32(4)], m)
    e2_hi = _lane_i32(pv[_i32(5)], m)
    c1 = pl.multiple_of(_lane_i32(pv[_i32(6)], m), 8)
    c2 = pl.multiple_of(_lane_i32(pv[_i32(7)], m), 8)
    inv = _lane_f32(invv[...], m)

    cp_e1 = pltpu.async_copy(
        x_hbm.at[pl.ds(c1, EDGE), pl.ds(col0, DH)], e1v, sem_e1)
    cp_e2 = pltpu.async_copy(
        x_hbm.at[pl.ds(c2, EDGE), pl.ds(col0, DH)], e2v, sem_e2)

    zero = tuple(jnp.zeros((L,), jnp.float32) for _ in range(DH // L))

    cp_bs.wait()

    def fb_body(bk, accs):
        return tuple(accs[c] + bsv[bk, pl.ds(c * L, L)]
                     for c in range(DH // L))

    accs = lax.fori_loop(fb_lo, fb_hi, fb_body, zero)

    cp_e1.wait()

    def e1_body(r, accs):
        return tuple(accs[c] + e1v[r, pl.ds(c * L, L)]
                     for c in range(DH // L))

    accs = lax.fori_loop(e1_lo - c1, e1_hi - c1, e1_body, accs)

    cp_e2.wait()

    def e2_body(r, accs):
        return tuple(accs[c] + e2v[r, pl.ds(c * L, L)]
                     for c in range(DH // L))

    accs = lax.fori_loop(e2_lo - c2, e2_hi - c2, e2_body, accs)

    for c in range(DH // L):
        outv[pl.ds(c * L, L)] = accs[c] * inv
    pltpu.sync_copy(outv, out_hbm.at[pl.ds(m * D + col0, DH)])


def kernel(atom_hiddens, a_scope):
    x = atom_hiddens.astype(jnp.float32)
    s = a_scope[:, 0].astype(jnp.int32)
    sz = a_scope[:, 1].astype(jnp.int32)
    e = jnp.minimum(s + sz, N)
    b0 = (s + BLK - 1) // BLK
    b1 = e // BLK
    has_full = b0 < b1
    fb_lo = jnp.where(has_full, b0, 0)
    fb_hi = jnp.where(has_full, b1, 0)
    e1_lo = s
    e1_hi = jnp.where(has_full, b0 * BLK, e)
    e2_lo = jnp.where(has_full, b1 * BLK, 0)
    e2_hi = jnp.where(has_full, e, 0)
    c1 = jnp.minimum((e1_lo // 8) * 8, N - EDGE)
    c2 = jnp.minimum((e2_lo // 8) * 8, N - EDGE)
    pi = jnp.stack([fb_lo, fb_hi, e1_lo, e1_hi, e2_lo, e2_hi, c1, c2])
    inv = jnp.where(sz > 0, 1.0 / jnp.maximum(sz, 1).astype(jnp.float32), 0.0)

    bs = _block_sums(x)
    return _combine(x, bs, pi, inv).reshape(B, D)
